# Initial kernel scaffold; baseline (speedup 1.0000x reference)
import jax
import jax.numpy as jnp
from jax.experimental import pallas as pl


def kernel(start, path, end, te, pe):
    raise NotImplementedError("write your pallas kernel here")


# speedup vs baseline: 3.8828x; 3.8828x over previous
import functools

import jax
import jax.numpy as jnp
from jax import lax
from jax.experimental import pallas as pl
from jax.experimental.pallas import tpu as pltpu
from jax.experimental.pallas import tpu_sc as plsc

_DIM = 64
_IDXW = 128
_GCHUNK = 10


def _gather_all(start_i, path_i, end_i, te, pe):
    n = start_i.shape[0]
    info = plsc.get_sparse_core_info()
    nw = info.num_cores * info.num_subcores
    chunk = _GCHUNK * _IDXW
    rows_per_w = n // nw
    n_chunks = rows_per_w // chunk

    mesh = plsc.VectorSubcoreMesh(core_axis_name="c", subcore_axis_name="s")

    @functools.partial(
        pl.kernel,
        out_type=jax.ShapeDtypeStruct((3, n, _DIM), jnp.float32),
        mesh=mesh,
        scratch_types=[
            pltpu.VMEM((chunk,), jnp.int32),
            pltpu.VMEM((chunk, _DIM), jnp.float32),
            pltpu.SemaphoreType.DMA,
        ],
        compiler_params=pltpu.CompilerParams(use_tc_tiling_on_sc=False),
    )
    def k(start_h, path_h, end_h, te_h, pe_h, out_h, idx_v, rows_v, sem):
        wid = lax.axis_index("s") * info.num_cores + lax.axis_index("c")
        rbase = wid * rows_per_w

        def do_table(row0, idx_h, tab_h, slot):
            pltpu.sync_copy(idx_h.at[pl.ds(row0, chunk)], idx_v)
            copies = []
            for g in range(_GCHUNK):
                copies.append(
                    pltpu.async_copy(
                        tab_h.at[idx_v.at[pl.ds(g * _IDXW, _IDXW)]],
                        rows_v.at[pl.ds(g * _IDXW, _IDXW)],
                        sem,
                    )
                )
            for c in copies:
                c.wait()
            pltpu.sync_copy(rows_v, out_h.at[slot, pl.ds(row0, chunk)])

        def chunk_body(j, carry):
            row0 = pl.multiple_of(rbase + j * chunk, chunk)
            do_table(row0, start_h, te_h, 0)
            do_table(row0, path_h, pe_h, 1)
            do_table(row0, end_h, te_h, 2)
            return carry

        lax.fori_loop(0, n_chunks, chunk_body, 0)

    return k(start_i, path_i, end_i, te, pe)


def kernel(start, path, end, te, pe):
    b, mp, _ = start.shape
    n = b * mp
    out3 = _gather_all(start.reshape(n), path.reshape(n), end.reshape(n), te, pe)
    return jnp.moveaxis(out3.reshape(3, b, mp, _DIM), 0, 2)
